# baseline (speedup 1.0000x reference)
import functools

import jax
import jax.numpy as jnp
from jax import lax
from jax.experimental import pallas as pl
from jax.experimental.pallas import tpu as pltpu
from jax.experimental.pallas import tpu_sc as plsc

_NUM_TOKENS = 32768
_HIDDEN = 768
_NUM_EXPERTS = 64
_TOP_K = 8

_ROW_BLOCK = 4096

_NUM_WORKERS = 32
_ROWS_PER_W = _NUM_TOKENS // _NUM_WORKERS
_LANES = 16
_GROUPS_PER_W = _ROWS_PER_W // _LANES


def _tc_body(x_ref, w_ref, logits_ref, probs_ref):
    x = x_ref[...]
    w = w_ref[...]
    logits = lax.dot_general(
        x, w, (((1,), (1,)), ((), ())), preferred_element_type=jnp.float32
    )
    m = jnp.max(logits, axis=1, keepdims=True)
    e = jnp.exp(logits - m)
    probs = e / jnp.sum(e, axis=1, keepdims=True)
    logits_ref[...] = logits
    probs_ref[...] = probs


def _tc_logits_probs(x, W):
    grid = (_NUM_TOKENS // _ROW_BLOCK,)
    out_shape = jax.ShapeDtypeStruct((_NUM_TOKENS, _NUM_EXPERTS), jnp.float32)
    return pl.pallas_call(
        _tc_body,
        grid=grid,
        in_specs=[
            pl.BlockSpec((_ROW_BLOCK, _HIDDEN), lambda i: (i, 0)),
            pl.BlockSpec((_NUM_EXPERTS, _HIDDEN), lambda i: (0, 0)),
        ],
        out_specs=[
            pl.BlockSpec((_ROW_BLOCK, _NUM_EXPERTS), lambda i: (i, 0)),
            pl.BlockSpec((_ROW_BLOCK, _NUM_EXPERTS), lambda i: (i, 0)),
        ],
        out_shape=[out_shape, out_shape],
        compiler_params=pltpu.CompilerParams(
            dimension_semantics=("arbitrary",)
        ),
    )(x, W)


def _sc_topk_body(logits_hbm, idx_hbm, val_hbm, blk_v, oi_v, ov_v):
    wid = lax.axis_index("s") * 2 + lax.axis_index("c")

    pltpu.sync_copy(
        logits_hbm.at[pl.ds(wid * _ROWS_PER_W * _NUM_EXPERTS,
                            _ROWS_PER_W * _NUM_EXPERTS)],
        blk_v,
    )

    lane = lax.iota(jnp.int32, _LANES)
    neg_inf = jnp.full((_LANES,), -jnp.inf, jnp.float32)
    _CHUNK = 16
    _NCHUNK = _NUM_EXPERTS // _CHUNK

    def _tourney(pairs):
        while len(pairs) > 1:
            nxt = []
            for j in range(0, len(pairs), 2):
                (lv, li), (hv, hi) = pairs[j], pairs[j + 1]
                gt = hv > lv
                nxt.append((jnp.where(gt, hv, lv), jnp.where(gt, hi, li)))
            pairs = nxt
        return pairs[0]

    def group_body(g, carry):
        row_base = (g * _LANES + lane) * _NUM_EXPERTS

        chunk_best = []
        for c in range(_NCHUNK):
            leaves = []
            for i in range(_CHUNK):
                e = c * _CHUNK + i
                col = jnp.full((_LANES,), e, jnp.int32)
                v = plsc.load_gather(blk_v, [row_base + col])
                leaves.append((v, col))
            chunk_best.append(_tourney(leaves))

        sel_vals = []
        sel_idxs = []
        for k in range(_TOP_K):
            wv, wi = _tourney(list(chunk_best))
            sel_vals.append(wv)
            sel_idxs.append(wi)
            if k + 1 == _TOP_K:
                break
            plsc.store_scatter(blk_v, [row_base + wi], neg_inf)
            cb = jnp.bitwise_and(wi, jnp.full((_LANES,), -_CHUNK, jnp.int32))
            gbase = row_base + cb
            leaves = []
            for i in range(_CHUNK):
                ic = jnp.full((_LANES,), i, jnp.int32)
                v = plsc.load_gather(blk_v, [gbase + ic])
                leaves.append((v, cb + ic))
            nv, ni = _tourney(leaves)
            for c in range(_NCHUNK):
                msk = cb == jnp.full((_LANES,), c * _CHUNK, jnp.int32)
                ov, oi = chunk_best[c]
                chunk_best[c] = (
                    jnp.where(msk, nv, ov),
                    jnp.where(msk, ni, oi),
                )

        top = sel_vals[0]
        exps = [jnp.exp(v - top) for v in sel_vals]
        total = exps[0]
        for p in exps[1:]:
            total = total + p

        out_base = (g * _LANES + lane) * _TOP_K
        for k in range(_TOP_K):
            kcol = jnp.full((_LANES,), k, jnp.int32)
            plsc.store_scatter(oi_v, [out_base + kcol], sel_idxs[k])
            plsc.store_scatter(ov_v, [out_base + kcol], exps[k] / total)
        return carry

    lax.fori_loop(0, _GROUPS_PER_W, group_body, 0, unroll=2)

    out_n = _ROWS_PER_W * _TOP_K
    pltpu.sync_copy(oi_v, idx_hbm.at[pl.ds(wid * out_n, out_n)])
    pltpu.sync_copy(ov_v, val_hbm.at[pl.ds(wid * out_n, out_n)])


def _sc_topk(logits):
    mesh = plsc.VectorSubcoreMesh(core_axis_name="c", subcore_axis_name="s")
    fn = functools.partial(
        pl.kernel,
        mesh=mesh,
        out_type=[
            jax.ShapeDtypeStruct((_NUM_TOKENS * _TOP_K,), jnp.int32),
            jax.ShapeDtypeStruct((_NUM_TOKENS * _TOP_K,), jnp.float32),
        ],
        scratch_types=[
            pltpu.VMEM((_ROWS_PER_W * _NUM_EXPERTS,), jnp.float32),
            pltpu.VMEM((_ROWS_PER_W * _TOP_K,), jnp.int32),
            pltpu.VMEM((_ROWS_PER_W * _TOP_K,), jnp.float32),
        ],
        compiler_params=pltpu.CompilerParams(needs_layout_passes=False),
    )(_sc_topk_body)
    idx_flat, val_flat = fn(logits.reshape(-1))
    return (
        idx_flat.reshape(_NUM_TOKENS, _TOP_K),
        val_flat.reshape(_NUM_TOKENS, _TOP_K),
    )


def kernel(x, W):
    logits, probs = _tc_logits_probs(x, W)
    top_k_indices, top_k_probs = _sc_topk(logits)
    return (logits, probs, top_k_indices, top_k_probs)


# speedup vs baseline: 1.0130x; 1.0130x over previous
import functools

import jax
import jax.numpy as jnp
from jax import lax
from jax.experimental import pallas as pl
from jax.experimental.pallas import tpu as pltpu
from jax.experimental.pallas import tpu_sc as plsc

_NUM_TOKENS = 32768
_HIDDEN = 768
_NUM_EXPERTS = 64
_TOP_K = 8

_ROW_BLOCK = 4096

_NUM_WORKERS = 32
_ROWS_PER_W = _NUM_TOKENS // _NUM_WORKERS
_LANES = 16
_GROUPS_PER_W = _ROWS_PER_W // _LANES


def _tc_body(x_ref, w_ref, logits_ref, probs_ref):
    x = x_ref[...]
    w = w_ref[...]
    logits = lax.dot_general(
        x, w, (((1,), (1,)), ((), ())), preferred_element_type=jnp.float32
    )
    m = jnp.max(logits, axis=1, keepdims=True)
    e = jnp.exp(logits - m)
    probs = e / jnp.sum(e, axis=1, keepdims=True)
    logits_ref[...] = logits
    probs_ref[...] = probs


def _tc_logits_probs(x, W):
    grid = (_NUM_TOKENS // _ROW_BLOCK,)
    out_shape = jax.ShapeDtypeStruct((_NUM_TOKENS, _NUM_EXPERTS), jnp.float32)
    return pl.pallas_call(
        _tc_body,
        grid=grid,
        in_specs=[
            pl.BlockSpec((_ROW_BLOCK, _HIDDEN), lambda i: (i, 0)),
            pl.BlockSpec((_NUM_EXPERTS, _HIDDEN), lambda i: (0, 0)),
        ],
        out_specs=[
            pl.BlockSpec((_ROW_BLOCK, _NUM_EXPERTS), lambda i: (i, 0)),
            pl.BlockSpec((_ROW_BLOCK, _NUM_EXPERTS), lambda i: (i, 0)),
        ],
        out_shape=[out_shape, out_shape],
        compiler_params=pltpu.CompilerParams(
            dimension_semantics=("arbitrary",)
        ),
    )(x, W)


def _sc_topk_body(logits_hbm, idx_hbm, val_hbm, blk_v, oi_v, ov_v):
    wid = lax.axis_index("s") * 2 + lax.axis_index("c")

    pltpu.sync_copy(
        logits_hbm.at[pl.ds(wid * _ROWS_PER_W * _NUM_EXPERTS,
                            _ROWS_PER_W * _NUM_EXPERTS)],
        blk_v,
    )

    lane = lax.iota(jnp.int32, _LANES)
    neg_inf = jnp.full((_LANES,), -jnp.inf, jnp.float32)
    _CHUNK = 16
    _NCHUNK = _NUM_EXPERTS // _CHUNK

    def _tourney(pairs):
        while len(pairs) > 1:
            nxt = []
            for j in range(0, len(pairs), 2):
                (lv, li), (hv, hi) = pairs[j], pairs[j + 1]
                gt = hv > lv
                nxt.append((jnp.where(gt, hv, lv), jnp.where(gt, hi, li)))
            pairs = nxt
        return pairs[0]

    @plsc.parallel_loop(0, _GROUPS_PER_W, unroll=2)
    def group_body(g):
        row_base = (g * _LANES + lane) * _NUM_EXPERTS

        chunk_best = []
        for c in range(_NCHUNK):
            leaves = []
            for i in range(_CHUNK):
                e = c * _CHUNK + i
                col = jnp.full((_LANES,), e, jnp.int32)
                v = plsc.load_gather(blk_v, [row_base + col])
                leaves.append((v, col))
            chunk_best.append(_tourney(leaves))

        sel_vals = []
        sel_idxs = []
        for k in range(_TOP_K):
            wv, wi = _tourney(list(chunk_best))
            sel_vals.append(wv)
            sel_idxs.append(wi)
            if k + 1 == _TOP_K:
                break
            plsc.store_scatter(blk_v, [row_base + wi], neg_inf)
            cb = jnp.bitwise_and(wi, jnp.full((_LANES,), -_CHUNK, jnp.int32))
            gbase = row_base + cb
            leaves = []
            for i in range(_CHUNK):
                ic = jnp.full((_LANES,), i, jnp.int32)
                v = plsc.load_gather(blk_v, [gbase + ic])
                leaves.append((v, cb + ic))
            nv, ni = _tourney(leaves)
            for c in range(_NCHUNK):
                msk = cb == jnp.full((_LANES,), c * _CHUNK, jnp.int32)
                ov, oi = chunk_best[c]
                chunk_best[c] = (
                    jnp.where(msk, nv, ov),
                    jnp.where(msk, ni, oi),
                )

        top = sel_vals[0]
        exps = [jnp.exp(v - top) for v in sel_vals]
        total = exps[0]
        for p in exps[1:]:
            total = total + p

        out_base = (g * _LANES + lane) * _TOP_K
        for k in range(_TOP_K):
            kcol = jnp.full((_LANES,), k, jnp.int32)
            plsc.store_scatter(oi_v, [out_base + kcol], sel_idxs[k])
            plsc.store_scatter(ov_v, [out_base + kcol], exps[k] / total)

    out_n = _ROWS_PER_W * _TOP_K
    pltpu.sync_copy(oi_v, idx_hbm.at[pl.ds(wid * out_n, out_n)])
    pltpu.sync_copy(ov_v, val_hbm.at[pl.ds(wid * out_n, out_n)])


def _sc_topk(logits):
    mesh = plsc.VectorSubcoreMesh(core_axis_name="c", subcore_axis_name="s")
    fn = functools.partial(
        pl.kernel,
        mesh=mesh,
        out_type=[
            jax.ShapeDtypeStruct((_NUM_TOKENS * _TOP_K,), jnp.int32),
            jax.ShapeDtypeStruct((_NUM_TOKENS * _TOP_K,), jnp.float32),
        ],
        scratch_types=[
            pltpu.VMEM((_ROWS_PER_W * _NUM_EXPERTS,), jnp.float32),
            pltpu.VMEM((_ROWS_PER_W * _TOP_K,), jnp.int32),
            pltpu.VMEM((_ROWS_PER_W * _TOP_K,), jnp.float32),
        ],
        compiler_params=pltpu.CompilerParams(needs_layout_passes=False),
    )(_sc_topk_body)
    idx_flat, val_flat = fn(logits.reshape(-1))
    return (
        idx_flat.reshape(_NUM_TOKENS, _TOP_K),
        val_flat.reshape(_NUM_TOKENS, _TOP_K),
    )


def kernel(x, W):
    logits, probs = _tc_logits_probs(x, W)
    top_k_indices, top_k_probs = _sc_topk(logits)
    return (logits, probs, top_k_indices, top_k_probs)
